# baseline (speedup 1.0000x reference)
import functools

import jax
import jax.numpy as jnp
from jax import lax
from jax.experimental import pallas as pl
from jax.experimental.pallas import tpu as pltpu
from jax.experimental.pallas import tpu_sc as plsc

_INFO = plsc.get_sparse_core_info()
_NC, _NS = _INFO.num_cores, _INFO.num_subcores
_NW = _NC * _NS


def kernel(x):
    B, T, C = x.shape
    rows = B * T
    x2 = x.reshape(rows, C)
    rpw = rows // _NW

    mesh = plsc.VectorSubcoreMesh(core_axis_name="c", subcore_axis_name="s")

    @functools.partial(
        pl.kernel,
        mesh=mesh,
        out_type=jax.ShapeDtypeStruct((rows, C), x.dtype),
    )
    def sc_copy(x_hbm, o_hbm):
        wid = lax.axis_index("s") * _NC + lax.axis_index("c")
        base = wid * rpw
        pltpu.sync_copy(x_hbm.at[pl.ds(base, rpw)], o_hbm.at[pl.ds(base, rpw)])

    return sc_copy(x2).reshape(B, T, C)


# speedup vs baseline: 34.8302x; 34.8302x over previous
import functools

import jax
import jax.numpy as jnp
from jax import lax
from jax.experimental import pallas as pl
from jax.experimental.pallas import tpu as pltpu
from jax.experimental.pallas import tpu_sc as plsc

_INFO = plsc.get_sparse_core_info()
_NC, _NS = _INFO.num_cores, _INFO.num_subcores
_NW = _NC * _NS

_CHUNK = 64


def kernel(x):
    B, T, C = x.shape
    rows = B * T
    x2 = x.reshape(rows, C)
    rpw = rows // _NW
    n_chunks = rpw // _CHUNK

    mesh = plsc.VectorSubcoreMesh(core_axis_name="c", subcore_axis_name="s")

    @functools.partial(
        pl.kernel,
        mesh=mesh,
        out_type=jax.ShapeDtypeStruct((rows, C), x.dtype),
        scratch_types=[
            pltpu.VMEM((_CHUNK, C), x.dtype),
            pltpu.VMEM((_CHUNK, C), x.dtype),
            pltpu.SemaphoreType.DMA,
            pltpu.SemaphoreType.DMA,
            pltpu.SemaphoreType.DMA,
            pltpu.SemaphoreType.DMA,
        ],
    )
    def sc_copy(x_hbm, o_hbm, buf0, buf1, in0, in1, out0, out1):
        wid = lax.axis_index("s") * _NC + lax.axis_index("c")
        base = wid * rpw
        bufs = (buf0, buf1)
        in_sems = (in0, in1)
        out_sems = (out0, out1)

        def load(i):
            s = i % 2
            pltpu.make_async_copy(
                x_hbm.at[pl.ds(base + i * _CHUNK, _CHUNK)], bufs[s], in_sems[s]
            ).start()

        def wait_load(i):
            s = i % 2
            pltpu.make_async_copy(
                x_hbm.at[pl.ds(base + i * _CHUNK, _CHUNK)], bufs[s], in_sems[s]
            ).wait()

        def store(i):
            s = i % 2
            pltpu.make_async_copy(
                bufs[s], o_hbm.at[pl.ds(base + i * _CHUNK, _CHUNK)], out_sems[s]
            ).start()

        def wait_store(i):
            s = i % 2
            pltpu.make_async_copy(
                bufs[s], o_hbm.at[pl.ds(base + i * _CHUNK, _CHUNK)], out_sems[s]
            ).wait()

        load(0)
        for i in range(n_chunks):
            if i + 1 < n_chunks:
                if i >= 1:
                    wait_store(i - 1)
                load(i + 1)
            wait_load(i)
            store(i)
        for i in range(max(n_chunks - 2, 0), n_chunks):
            wait_store(i)

    return sc_copy(x2).reshape(B, T, C)
